# baseline (speedup 1.0000x reference)
import functools

import jax
import jax.numpy as jnp
from jax import lax
from jax.experimental import pallas as pl
from jax.experimental.pallas import tpu as pltpu
from jax.experimental.pallas import tpu_sc as plsc

N = 10000
D = 128
E = 320000
NC = 2
NS = 16
NW = NC * NS
CHUNK = 128
CPT = 80
PH = 2
CPH = CPT // PH
E_PAD = NW * CPT * CHUNK
N_PAD = 10240
RPT = N_PAD // NS
BLK = 640
DUMMY = N

_mesh = plsc.VectorSubcoreMesh(core_axis_name="c", subcore_axis_name="s")


@functools.partial(
    pl.kernel,
    out_type=jax.ShapeDtypeStruct((NC * N_PAD, D), jnp.float32),
    mesh=_mesh,
    scratch_types=[
        pltpu.VMEM((CPT, CHUNK), jnp.int32),
        pltpu.VMEM((CHUNK, D), jnp.float32),
        pltpu.VMEM_SHARED((N_PAD, D), jnp.float32),
    ],
)
def _sc_degree(col_hbm, deg_hbm, col_v, ones_v, deg_sh):
    cid = lax.axis_index("c")
    sid = lax.axis_index("s")
    g = cid * NS + sid
    pltpu.sync_copy(col_hbm.at[g], col_v)

    @pl.loop(0, CHUNK)
    def _(i):
        @pl.loop(0, D, step=16)
        def _(k):
            ones_v[i, pl.ds(k, 16)] = jnp.zeros((16,), jnp.float32)

    @pl.loop(0, RPT, step=CHUNK)
    def _(r):
        pltpu.sync_copy(ones_v, deg_sh.at[pl.ds(sid * RPT + r, CHUNK)])

    @pl.loop(0, CHUNK)
    def _(i):
        @pl.loop(0, D, step=16)
        def _(k):
            ones_v[i, pl.ds(k, 16)] = jnp.ones((16,), jnp.float32)

    plsc.subcore_barrier()

    @pl.loop(0, CPT)
    def _(j):
        pltpu.sync_copy(ones_v, deg_sh.at[col_v.at[j]], add=True)

    plsc.subcore_barrier()
    pltpu.sync_copy(deg_sh.at[pl.ds(sid * RPT, RPT)],
                    deg_hbm.at[pl.ds(cid * N_PAD + sid * RPT, RPT)])


@functools.partial(
    pl.kernel,
    out_type=jax.ShapeDtypeStruct((NC * N_PAD, D), jnp.float32),
    mesh=_mesh,
    scratch_types=[
        pltpu.VMEM((CPH, CHUNK), jnp.int32),
        pltpu.VMEM((CPH, CHUNK), jnp.int32),
        pltpu.VMEM((CHUNK, D), jnp.float32),
        pltpu.VMEM((CHUNK, D), jnp.float32),
        pltpu.SemaphoreType.DMA,
        pltpu.SemaphoreType.DMA,
        pltpu.SemaphoreType.DMA,
        pltpu.SemaphoreType.DMA,
        pltpu.VMEM_SHARED((N_PAD, D), jnp.float32),
    ],
)
def _sc_scatter(y_hbm, row_hbm, col_hbm, part_hbm, row_v, col_v,
                buf_a, buf_b, sem_a, sem_a2, sem_b, sem_b2, acc):
    cid = lax.axis_index("c")
    sid = lax.axis_index("s")
    g = cid * NS + sid

    @pl.loop(0, CHUNK)
    def _(i):
        @pl.loop(0, D, step=16)
        def _(k):
            buf_a[i, pl.ds(k, 16)] = jnp.zeros((16,), jnp.float32)

    @pl.loop(0, RPT, step=CHUNK)
    def _(r):
        pltpu.sync_copy(buf_a, acc.at[pl.ds(sid * RPT + r, CHUNK)])

    plsc.subcore_barrier()

    for h in range(PH):
        pltpu.sync_copy(row_hbm.at[g * PH + h], row_v)
        pltpu.sync_copy(col_hbm.at[g * PH + h], col_v)

        @pl.loop(0, CPH, step=2)
        def _(k):
            cp_a = pltpu.async_copy(y_hbm.at[row_v.at[k]], buf_a, sem_a)
            cp_b = pltpu.async_copy(y_hbm.at[row_v.at[k + 1]], buf_b, sem_b)
            cp_a.wait()
            cs_a = pltpu.async_copy(buf_a, acc.at[col_v.at[k]], sem_a2,
                                    add=True)
            cp_b.wait()
            cs_b = pltpu.async_copy(buf_b, acc.at[col_v.at[k + 1]], sem_b2,
                                    add=True)
            cs_a.wait()
            cs_b.wait()

    plsc.subcore_barrier()
    pltpu.sync_copy(acc.at[pl.ds(sid * RPT, RPT)],
                    part_hbm.at[pl.ds(cid * N_PAD + sid * RPT, RPT)])


def _norm_body(degp_ref, x_ref, w_ref, y_ref, dinv_ref):
    deg = degp_ref[0, :, 0] + degp_ref[1, :, 0] + 1.0
    dinv = lax.rsqrt(deg)
    dinv_ref[0, :] = dinv
    xw = jnp.dot(x_ref[...], w_ref[...], preferred_element_type=jnp.float32)
    y_ref[...] = xw * dinv[:, None]


_norm = pl.pallas_call(
    _norm_body,
    grid=(N_PAD // BLK,),
    in_specs=[pl.BlockSpec((NC, BLK, D), lambda i: (0, i, 0)),
              pl.BlockSpec((BLK, D), lambda i: (i, 0)),
              pl.BlockSpec((D, D), lambda i: (0, 0))],
    out_specs=[pl.BlockSpec((BLK, D), lambda i: (i, 0)),
               pl.BlockSpec((1, BLK), lambda i: (0, i))],
    out_shape=[jax.ShapeDtypeStruct((N_PAD, D), jnp.float32),
               jax.ShapeDtypeStruct((1, N_PAD), jnp.float32)],
)


def _layer_body(p_ref, y_ref, dinv_ref, b_ref, w_ref, y2_ref):
    arr = p_ref[...]
    dinv = dinv_ref[0, :][:, None]
    pre = (arr[0] + arr[1] + y_ref[...]) * dinv + b_ref[...]
    h = jnp.maximum(pre, 0.0)
    y2_ref[...] = jnp.dot(h, w_ref[...], preferred_element_type=jnp.float32) * dinv


_layer = pl.pallas_call(
    _layer_body,
    grid=(N_PAD // BLK,),
    in_specs=[pl.BlockSpec((NC, BLK, D), lambda i: (0, i, 0)),
              pl.BlockSpec((BLK, D), lambda i: (i, 0)),
              pl.BlockSpec((1, BLK), lambda i: (0, i)),
              pl.BlockSpec((1, D), lambda i: (0, 0)),
              pl.BlockSpec((D, D), lambda i: (0, 0))],
    out_specs=pl.BlockSpec((BLK, D), lambda i: (i, 0)),
    out_shape=jax.ShapeDtypeStruct((N_PAD, D), jnp.float32),
)


def _final_body(p_ref, y_ref, dinv_ref, b_ref, o_ref):
    arr = p_ref[...]
    dinv = dinv_ref[0, :][:, None]
    o_ref[...] = (arr[0] + arr[1] + y_ref[...]) * dinv + b_ref[...]


_final = pl.pallas_call(
    _final_body,
    grid=(N_PAD // BLK,),
    in_specs=[pl.BlockSpec((NC, BLK, D), lambda i: (0, i, 0)),
              pl.BlockSpec((BLK, D), lambda i: (i, 0)),
              pl.BlockSpec((1, BLK), lambda i: (0, i)),
              pl.BlockSpec((1, D), lambda i: (0, 0))],
    out_specs=pl.BlockSpec((BLK, D), lambda i: (i, 0)),
    out_shape=jax.ShapeDtypeStruct((N_PAD, D), jnp.float32),
)


def kernel(x, edge_index, W1, b1, W2, b2):
    ei = edge_index.astype(jnp.int32)
    padv = DUMMY + (jnp.arange(E_PAD - E, dtype=jnp.int32) % (N_PAD - N))
    row = jnp.concatenate([ei[0], padv]).reshape(NW, CPT, CHUNK)
    col = jnp.concatenate([ei[1], padv]).reshape(NW, CPT, CHUNK)
    srow = row.reshape(NW * PH, CPH, CHUNK)
    scol = col.reshape(NW * PH, CPH, CHUNK)
    xp = jnp.pad(x, ((0, N_PAD - N), (0, 0)))
    b1r = b1.reshape(1, D)
    b2r = b2.reshape(1, D)

    degp = _sc_degree(col).reshape(NC, N_PAD, D)
    y1, dinv = _norm(degp, xp, W1)
    p1 = _sc_scatter(y1, srow, scol).reshape(NC, N_PAD, D)
    y2 = _layer(p1, y1, dinv, b1r, W2)
    p2 = _sc_scatter(y2, srow, scol).reshape(NC, N_PAD, D)
    out = _final(p2, y2, dinv, b2r)
    return out[:N]


# speedup vs baseline: 1.0004x; 1.0004x over previous
import functools

import jax
import jax.numpy as jnp
from jax import lax
from jax.experimental import pallas as pl
from jax.experimental.pallas import tpu as pltpu
from jax.experimental.pallas import tpu_sc as plsc

N = 10000
D = 128
E = 320000
NC = 2
NS = 16
NW = NC * NS
CHUNK = 128
CPT = 80
PH = 2
CPH = CPT // PH
E_PAD = NW * CPT * CHUNK
N_PAD = 10240
RPT = N_PAD // NS
BLK = 640
DUMMY = N

_mesh = plsc.VectorSubcoreMesh(core_axis_name="c", subcore_axis_name="s")


@functools.partial(
    pl.kernel,
    out_type=jax.ShapeDtypeStruct((NC * N_PAD, D), jnp.float32),
    mesh=_mesh,
    scratch_types=[
        pltpu.VMEM((CPT, CHUNK), jnp.int32),
        pltpu.VMEM((CHUNK, D), jnp.float32),
        pltpu.SemaphoreType.DMA,
        pltpu.SemaphoreType.DMA,
        pltpu.VMEM_SHARED((N_PAD, D), jnp.float32),
    ],
)
def _sc_degree(col_hbm, deg_hbm, col_v, ones_v, sem_a, sem_b, deg_sh):
    cid = lax.axis_index("c")
    sid = lax.axis_index("s")
    g = cid * NS + sid
    pltpu.sync_copy(col_hbm.at[g], col_v)

    @pl.loop(0, CHUNK)
    def _(i):
        @pl.loop(0, D, step=16)
        def _(k):
            ones_v[i, pl.ds(k, 16)] = jnp.zeros((16,), jnp.float32)

    @pl.loop(0, RPT, step=CHUNK)
    def _(r):
        pltpu.sync_copy(ones_v, deg_sh.at[pl.ds(sid * RPT + r, CHUNK)])

    @pl.loop(0, CHUNK)
    def _(i):
        @pl.loop(0, D, step=16)
        def _(k):
            ones_v[i, pl.ds(k, 16)] = jnp.ones((16,), jnp.float32)

    plsc.subcore_barrier()

    @pl.loop(0, CPT, step=2)
    def _(j):
        c1 = pltpu.async_copy(ones_v, deg_sh.at[col_v.at[j]], sem_a, add=True)
        c2 = pltpu.async_copy(ones_v, deg_sh.at[col_v.at[j + 1]], sem_b,
                              add=True)
        c1.wait()
        c2.wait()

    plsc.subcore_barrier()
    pltpu.sync_copy(deg_sh.at[pl.ds(sid * RPT, RPT)],
                    deg_hbm.at[pl.ds(cid * N_PAD + sid * RPT, RPT)])


@functools.partial(
    pl.kernel,
    out_type=jax.ShapeDtypeStruct((NC * N_PAD, D), jnp.float32),
    mesh=_mesh,
    scratch_types=[
        pltpu.VMEM((CPH, CHUNK), jnp.int32),
        pltpu.VMEM((CPH, CHUNK), jnp.int32),
        pltpu.VMEM((CHUNK, D), jnp.float32),
        pltpu.VMEM((CHUNK, D), jnp.float32),
        pltpu.SemaphoreType.DMA,
        pltpu.SemaphoreType.DMA,
        pltpu.SemaphoreType.DMA,
        pltpu.SemaphoreType.DMA,
        pltpu.VMEM_SHARED((N_PAD, D), jnp.float32),
    ],
)
def _sc_scatter(y_hbm, row_hbm, col_hbm, part_hbm, row_v, col_v,
                buf_a, buf_b, sem_a, sem_a2, sem_b, sem_b2, acc):
    cid = lax.axis_index("c")
    sid = lax.axis_index("s")
    g = cid * NS + sid

    @pl.loop(0, CHUNK)
    def _(i):
        @pl.loop(0, D, step=16)
        def _(k):
            buf_a[i, pl.ds(k, 16)] = jnp.zeros((16,), jnp.float32)

    @pl.loop(0, RPT, step=CHUNK)
    def _(r):
        pltpu.sync_copy(buf_a, acc.at[pl.ds(sid * RPT + r, CHUNK)])

    plsc.subcore_barrier()

    for h in range(PH):
        pltpu.sync_copy(row_hbm.at[g * PH + h], row_v)
        pltpu.sync_copy(col_hbm.at[g * PH + h], col_v)

        @pl.loop(0, CPH, step=2)
        def _(k):
            cp_a = pltpu.async_copy(y_hbm.at[row_v.at[k]], buf_a, sem_a)
            cp_b = pltpu.async_copy(y_hbm.at[row_v.at[k + 1]], buf_b, sem_b)
            cp_a.wait()
            cs_a = pltpu.async_copy(buf_a, acc.at[col_v.at[k]], sem_a2,
                                    add=True)
            cp_b.wait()
            cs_b = pltpu.async_copy(buf_b, acc.at[col_v.at[k + 1]], sem_b2,
                                    add=True)
            cs_a.wait()
            cs_b.wait()

    plsc.subcore_barrier()
    pltpu.sync_copy(acc.at[pl.ds(sid * RPT, RPT)],
                    part_hbm.at[pl.ds(cid * N_PAD + sid * RPT, RPT)])


def _norm_body(degp_ref, x_ref, w_ref, y_ref, dinv_ref):
    deg = degp_ref[0, :, 0] + degp_ref[1, :, 0] + 1.0
    dinv = lax.rsqrt(deg)
    dinv_ref[0, :] = dinv
    xw = jnp.dot(x_ref[...], w_ref[...], preferred_element_type=jnp.float32)
    y_ref[...] = xw * dinv[:, None]


_norm = pl.pallas_call(
    _norm_body,
    grid=(N_PAD // BLK,),
    in_specs=[pl.BlockSpec((NC, BLK, D), lambda i: (0, i, 0)),
              pl.BlockSpec((BLK, D), lambda i: (i, 0)),
              pl.BlockSpec((D, D), lambda i: (0, 0))],
    out_specs=[pl.BlockSpec((BLK, D), lambda i: (i, 0)),
               pl.BlockSpec((1, BLK), lambda i: (0, i))],
    out_shape=[jax.ShapeDtypeStruct((N_PAD, D), jnp.float32),
               jax.ShapeDtypeStruct((1, N_PAD), jnp.float32)],
)


def _layer_body(p_ref, y_ref, dinv_ref, b_ref, w_ref, y2_ref):
    arr = p_ref[...]
    dinv = dinv_ref[0, :][:, None]
    pre = (arr[0] + arr[1] + y_ref[...]) * dinv + b_ref[...]
    h = jnp.maximum(pre, 0.0)
    y2_ref[...] = jnp.dot(h, w_ref[...], preferred_element_type=jnp.float32) * dinv


_layer = pl.pallas_call(
    _layer_body,
    grid=(N_PAD // BLK,),
    in_specs=[pl.BlockSpec((NC, BLK, D), lambda i: (0, i, 0)),
              pl.BlockSpec((BLK, D), lambda i: (i, 0)),
              pl.BlockSpec((1, BLK), lambda i: (0, i)),
              pl.BlockSpec((1, D), lambda i: (0, 0)),
              pl.BlockSpec((D, D), lambda i: (0, 0))],
    out_specs=pl.BlockSpec((BLK, D), lambda i: (i, 0)),
    out_shape=jax.ShapeDtypeStruct((N_PAD, D), jnp.float32),
)


def _final_body(p_ref, y_ref, dinv_ref, b_ref, o_ref):
    arr = p_ref[...]
    dinv = dinv_ref[0, :][:, None]
    o_ref[...] = (arr[0] + arr[1] + y_ref[...]) * dinv + b_ref[...]


_final = pl.pallas_call(
    _final_body,
    grid=(N_PAD // BLK,),
    in_specs=[pl.BlockSpec((NC, BLK, D), lambda i: (0, i, 0)),
              pl.BlockSpec((BLK, D), lambda i: (i, 0)),
              pl.BlockSpec((1, BLK), lambda i: (0, i)),
              pl.BlockSpec((1, D), lambda i: (0, 0))],
    out_specs=pl.BlockSpec((BLK, D), lambda i: (i, 0)),
    out_shape=jax.ShapeDtypeStruct((N_PAD, D), jnp.float32),
)


def kernel(x, edge_index, W1, b1, W2, b2):
    ei = edge_index.astype(jnp.int32)
    padv = DUMMY + (jnp.arange(E_PAD - E, dtype=jnp.int32) % (N_PAD - N))
    row = jnp.concatenate([ei[0], padv]).reshape(NW, CPT, CHUNK)
    col = jnp.concatenate([ei[1], padv]).reshape(NW, CPT, CHUNK)
    srow = row.reshape(NW * PH, CPH, CHUNK)
    scol = col.reshape(NW * PH, CPH, CHUNK)
    xp = jnp.pad(x, ((0, N_PAD - N), (0, 0)))
    b1r = b1.reshape(1, D)
    b2r = b2.reshape(1, D)

    degp = _sc_degree(col).reshape(NC, N_PAD, D)
    y1, dinv = _norm(degp, xp, W1)
    p1 = _sc_scatter(y1, srow, scol).reshape(NC, N_PAD, D)
    y2 = _layer(p1, y1, dinv, b1r, W2)
    p2 = _sc_scatter(y2, srow, scol).reshape(NC, N_PAD, D)
    out = _final(p2, y2, dinv, b2r)
    return out[:N]
